# Initial kernel scaffold; baseline (speedup 1.0000x reference)
import jax
import jax.numpy as jnp
from jax.experimental import pallas as pl


def kernel(x0, x1, x2, w, b):
    raise NotImplementedError("write your pallas kernel here")


# speedup vs baseline: 2.2171x; 2.2171x over previous
import jax
import jax.numpy as jnp
from jax.experimental import pallas as pl
from jax.experimental.pallas import tpu as pltpu

_INV_SQRT2 = 0.7071067811865476
_C1, _C2, _C3, _C4, _C5 = 0.254829592, -0.284496736, 1.421413741, -1.453152027, 1.061405429
_CP = 0.3275911

_TM = 512


def _erf_gelu(y):
    x = y * _INV_SQRT2
    ax = jnp.abs(x)
    ex = jnp.exp(-ax * ax)
    d = 1.0 + _CP * ax
    r = pl.reciprocal(d, approx=True)
    r = r * (2.0 - d * r)
    poly = ((((_C5 * r + _C4) * r + _C3) * r + _C2) * r + _C1) * r
    erf = jnp.sign(x) * (1.0 - poly * ex)
    return 0.5 * y * (1.0 + erf)


def _body(x0_ref, x1_ref, x2_ref, e1_ref, e2_ref, w_ref, b_ref, o_ref):
    w = w_ref[...]
    f = w.shape[1]
    y0 = jnp.dot(x0_ref[...].astype(jnp.bfloat16), w[0:f, :],
                 preferred_element_type=jnp.float32)
    y1 = jnp.dot(x1_ref[...].astype(jnp.bfloat16), w[f:2 * f, :],
                 preferred_element_type=jnp.float32)
    y2 = jnp.dot(x2_ref[...].astype(jnp.bfloat16), w[2 * f:3 * f, :],
                 preferred_element_type=jnp.float32)
    acc = y0
    acc += jnp.dot(e1_ref[...], y1.astype(jnp.bfloat16),
                   preferred_element_type=jnp.float32)
    acc += jnp.dot(e2_ref[...], y2.astype(jnp.bfloat16),
                   preferred_element_type=jnp.float32)
    o_ref[...] = _erf_gelu(acc + b_ref[...])


def _expand_mat(tm, s):
    r = jax.lax.broadcasted_iota(jnp.int32, (tm, tm >> s), 0)
    j = jax.lax.broadcasted_iota(jnp.int32, (tm, tm >> s), 1)
    return ((r >> s) == j).astype(jnp.bfloat16)


def kernel(x0, x1, x2, w, b):
    batch, t, f = x0.shape
    rows = batch * t
    xs = [x0.reshape(rows, f),
          x1[:, :t >> 1, :].reshape(rows >> 1, f),
          x2[:, :t >> 2, :].reshape(rows >> 2, f)]

    tm = _TM
    grid = (rows // tm,)
    e1 = _expand_mat(tm, 1)
    e2 = _expand_mat(tm, 2)

    out = pl.pallas_call(
        _body,
        out_shape=jax.ShapeDtypeStruct((rows, f), x0.dtype),
        grid=grid,
        in_specs=[
            pl.BlockSpec((tm, f), lambda i: (i, 0)),
            pl.BlockSpec((tm >> 1, f), lambda i: (i, 0)),
            pl.BlockSpec((tm >> 2, f), lambda i: (i, 0)),
            pl.BlockSpec((tm, tm >> 1), lambda i: (0, 0)),
            pl.BlockSpec((tm, tm >> 2), lambda i: (0, 0)),
            pl.BlockSpec((3 * f, f), lambda i: (0, 0)),
            pl.BlockSpec((1, f), lambda i: (0, 0)),
        ],
        out_specs=pl.BlockSpec((tm, f), lambda i: (i, 0)),
        compiler_params=pltpu.CompilerParams(
            dimension_semantics=("parallel",)),
    )(*xs, e1, e2, w.astype(jnp.bfloat16), b)
    return out.reshape(batch, t, f)
